# baseline (speedup 1.0000x reference)
import functools

import jax
import jax.numpy as jnp
from jax import lax
from jax.experimental import pallas as pl
from jax.experimental.pallas import tpu as pltpu
from jax.experimental.pallas import tpu_sc as plsc

VOCAB = 1000000
D = 64
SCALE = 8.0

NC = 2
NS = 16
NW = NC * NS

AB = 128


def _make_emb(A: int, B: int):
    n_items = (A // AB) * B
    assert n_items % NW == 0
    per_w = n_items // NW
    n_at = A // AB
    mesh = plsc.VectorSubcoreMesh(
        core_axis_name="c", subcore_axis_name="s", num_cores=NC, num_subcores=NS
    )

    @functools.partial(
        pl.kernel,
        mesh=mesh,
        out_type=jax.ShapeDtypeStruct((B, D, A), jnp.float32),
        scratch_types=[
            pltpu.VMEM((AB,), jnp.int32),
            pltpu.VMEM((AB,), jnp.int32),
            pltpu.VMEM((AB,), jnp.int32),
            pltpu.VMEM((AB, 2 * D), jnp.float32),
            pltpu.VMEM((D, AB), jnp.float32),
            pltpu.SemaphoreType.DMA,
        ],
        compiler_params=pltpu.CompilerParams(needs_layout_passes=False),
    )
    def emb(xt_hbm, table2_hbm, out_hbm, idx_v, idx2_v, half_v, rows_v, outb,
            gsem):
        wid = lax.axis_index("s") * NC + lax.axis_index("c")
        lane = lax.iota(jnp.int32, 16)

        def item_body(t, carry):
            m = per_w * wid + t
            b = m // n_at
            at = m % n_at

            pltpu.sync_copy(xt_hbm.at[b, pl.ds(at * AB, AB)], idx_v)

            def prep(k, c2):
                v = idx_v[pl.ds(k * 16, 16)]
                idx2_v[pl.ds(k * 16, 16)] = v >> 1
                half_v[pl.ds(k * 16, 16)] = (v & 1) * D
                return c2

            lax.fori_loop(0, AB // 16, prep, 0, unroll=True)

            pltpu.async_copy(table2_hbm.at[idx2_v], rows_v, gsem).wait()

            def jbody(j, c2):
                for kb in range(AB // 16):
                    rows16 = lane + kb * 16
                    cols16 = half_v[pl.ds(kb * 16, 16)] + j
                    vals = plsc.load_gather(rows_v, [rows16, cols16])
                    outb[j, pl.ds(kb * 16, 16)] = vals
                return c2

            lax.fori_loop(0, D, jbody, 0)

            pltpu.sync_copy(outb, out_hbm.at[b, :, pl.ds(at * AB, AB)])
            return carry

        lax.fori_loop(0, per_w, item_body, 0)

    return emb


def kernel(x, table):
    A, B = x.shape
    xt = x.T
    table2 = (table * SCALE).reshape(VOCAB // 2, 2 * D)
    out3 = _make_emb(A, B)(xt, table2)
    return out3.transpose(2, 0, 1)


# speedup vs baseline: 2.0515x; 2.0515x over previous
import functools

import jax
import jax.numpy as jnp
from jax import lax
from jax.experimental import pallas as pl
from jax.experimental.pallas import tpu as pltpu
from jax.experimental.pallas import tpu_sc as plsc

VOCAB = 1000000
D = 64
SCALE = 8.0

NC = 2
NS = 16
NW = NC * NS

CHUNK = 128


def _make_emb(B: int):
    assert B % (NW * 2 * CHUNK) == 0
    b_per_w = B // NW
    n_chunks = b_per_w // CHUNK
    mesh = plsc.VectorSubcoreMesh(
        core_axis_name="c", subcore_axis_name="s", num_cores=NC, num_subcores=NS
    )

    @functools.partial(
        pl.kernel,
        mesh=mesh,
        out_type=jax.ShapeDtypeStruct((B, D), jnp.float32),
        scratch_types=[
            pltpu.VMEM((CHUNK,), jnp.int32),
            pltpu.VMEM((CHUNK,), jnp.int32),
            pltpu.VMEM((CHUNK,), jnp.int32),
            pltpu.VMEM((CHUNK,), jnp.int32),
            pltpu.VMEM((CHUNK, 2 * D), jnp.float32),
            pltpu.VMEM((CHUNK, 2 * D), jnp.float32),
            pltpu.VMEM((CHUNK, D), jnp.float32),
            pltpu.VMEM((CHUNK, D), jnp.float32),
            pltpu.SemaphoreType.DMA,
            pltpu.SemaphoreType.DMA,
            pltpu.SemaphoreType.DMA,
            pltpu.SemaphoreType.DMA,
        ],
        compiler_params=pltpu.CompilerParams(needs_layout_passes=False),
    )
    def emb(idx_hbm, table2_hbm, out_hbm, idx0, idx1, pair0, pair1,
            rows0, rows1, outv0, outv1,
            gsem0, gsem1, osem0, osem1):
        wid = lax.axis_index("s") * NC + lax.axis_index("c")
        wbase = wid * b_per_w

        idx_v = (idx0, idx1)
        pair_v = (pair0, pair1)
        rows_v = (rows0, rows1)
        outv = (outv0, outv1)
        gsem = (gsem0, gsem1)
        osem = (osem0, osem1)

        def stage_and_gather(g, b):
            base = wbase + g * CHUNK
            pltpu.sync_copy(idx_hbm.at[pl.ds(base, CHUNK)], idx_v[b])

            def prep(k, c2):
                v = idx_v[b][pl.ds(k * 16, 16)]
                pair_v[b][pl.ds(k * 16, 16)] = v >> 1
                return c2

            lax.fori_loop(0, CHUNK // 16, prep, 0, unroll=True)
            pltpu.async_copy(table2_hbm.at[pair_v[b]], rows_v[b], gsem[b])

        def select_scale(b):
            def grp_body(gg, c2):
                base = gg * 16
                hv = (idx_v[b][pl.ds(base, 16)] & 1) * D
                for i in range(16):
                    h = hv[i]
                    r = base + i
                    for k in range(D // 16):
                        outv[b][r, pl.ds(k * 16, 16)] = (
                            rows_v[b][r, pl.ds(h + k * 16, 16)] * SCALE
                        )
                return c2

            lax.fori_loop(0, CHUNK // 16, grp_body, 0)

        stage_and_gather(0, 0)

        def pair_body(g0, carry):
            for b in (0, 1):
                g = g0 + b
                nb = 1 - b

                @pl.when(g >= 1)
                def _():
                    pltpu.make_async_copy(
                        outv[nb],
                        out_hbm.at[pl.ds(wbase + (g - 1) * CHUNK, CHUNK)],
                        osem[nb],
                    ).wait()

                @pl.when(g + 1 < n_chunks)
                def _():
                    stage_and_gather(g + 1, nb)

                pltpu.make_async_copy(
                    table2_hbm.at[pair_v[b]], rows_v[b], gsem[b]
                ).wait()
                select_scale(b)
                pltpu.async_copy(
                    outv[b], out_hbm.at[pl.ds(wbase + g * CHUNK, CHUNK)], osem[b]
                )
            return carry

        lax.fori_loop(0, n_chunks // 2, lambda t, c: pair_body(t * 2, c), 0)

        pltpu.make_async_copy(
            outv[1],
            out_hbm.at[pl.ds(wbase + (n_chunks - 1) * CHUNK, CHUNK)],
            osem[1],
        ).wait()

    return emb


def kernel(x, table):
    S0, S1 = x.shape
    B = S0 * S1
    idx = x.reshape(B)
    table2 = table.reshape(VOCAB // 2, 2 * D)
    out = _make_emb(B)(idx, table2)
    return out.reshape(S0, S1, D)


# speedup vs baseline: 2.4819x; 1.2098x over previous
import functools

import jax
import jax.numpy as jnp
from jax import lax
from jax.experimental import pallas as pl
from jax.experimental.pallas import tpu as pltpu
from jax.experimental.pallas import tpu_sc as plsc

VOCAB = 1000000
D = 64
SCALE = 8.0

NC = 2
NS = 16
NW = NC * NS

CHUNK = 512


def _make_emb(B: int):
    assert B % (NW * 2 * CHUNK) == 0
    b_per_w = B // NW
    n_chunks = b_per_w // CHUNK
    mesh = plsc.VectorSubcoreMesh(
        core_axis_name="c", subcore_axis_name="s", num_cores=NC, num_subcores=NS
    )

    @functools.partial(
        pl.kernel,
        mesh=mesh,
        out_type=jax.ShapeDtypeStruct((B, D), jnp.float32),
        scratch_types=[
            pltpu.VMEM((CHUNK,), jnp.int32),
            pltpu.VMEM((CHUNK,), jnp.int32),
            pltpu.VMEM((CHUNK, D), jnp.float32),
            pltpu.VMEM((CHUNK, D), jnp.float32),
            pltpu.SemaphoreType.DMA,
            pltpu.SemaphoreType.DMA,
            pltpu.SemaphoreType.DMA,
            pltpu.SemaphoreType.DMA,
        ],
        compiler_params=pltpu.CompilerParams(use_tc_tiling_on_sc=False),
    )
    def emb(idx_hbm, table_hbm, out_hbm, idx0, idx1, rows0, rows1,
            gsem0, gsem1, osem0, osem1):
        wid = lax.axis_index("s") * NC + lax.axis_index("c")
        wbase = wid * b_per_w

        idx_v = (idx0, idx1)
        rows_v = (rows0, rows1)
        gsem = (gsem0, gsem1)
        osem = (osem0, osem1)

        def stage_and_gather(g, b):
            base = wbase + g * CHUNK
            pltpu.sync_copy(idx_hbm.at[pl.ds(base, CHUNK)], idx_v[b])
            pltpu.async_copy(table_hbm.at[idx_v[b]], rows_v[b], gsem[b])

        def scale_buf(b):
            def scale_row(i, c2):
                for j in range(D // 16):
                    sl = (i, pl.ds(j * 16, 16))
                    rows_v[b][sl] = rows_v[b][sl] * SCALE
                return c2

            lax.fori_loop(0, CHUNK, scale_row, 0, unroll=4)

        stage_and_gather(0, 0)

        def pair_body(g0, carry):
            for b in (0, 1):
                g = g0 + b
                nb = 1 - b

                @pl.when(g >= 1)
                def _():
                    pltpu.make_async_copy(
                        rows_v[nb],
                        out_hbm.at[pl.ds(wbase + (g - 1) * CHUNK, CHUNK)],
                        osem[nb],
                    ).wait()

                @pl.when(g + 1 < n_chunks)
                def _():
                    stage_and_gather(g + 1, nb)

                pltpu.make_async_copy(
                    table_hbm.at[idx_v[b]], rows_v[b], gsem[b]
                ).wait()
                scale_buf(b)
                pltpu.async_copy(
                    rows_v[b], out_hbm.at[pl.ds(wbase + g * CHUNK, CHUNK)], osem[b]
                )
            return carry

        lax.fori_loop(0, n_chunks // 2, lambda t, c: pair_body(t * 2, c), 0)

        pltpu.make_async_copy(
            rows_v[1],
            out_hbm.at[pl.ds(wbase + (n_chunks - 1) * CHUNK, CHUNK)],
            osem[1],
        ).wait()

    return emb


def kernel(x, table):
    S0, S1 = x.shape
    B = S0 * S1
    idx = x.reshape(B)
    out = _make_emb(B)(idx, table)
    return out.reshape(S0, S1, D)
